# baseline (speedup 1.0000x reference)
import functools

import jax
import jax.numpy as jnp
from jax import lax
from jax.experimental import pallas as pl
from jax.experimental.pallas import tpu as pltpu
from jax.experimental.pallas import tpu_sc as plsc

B = 16384
L = 64
V = 1000000
E = 64
H = 128
IMG = 2048
C = 1000


_MEAN_ROWS = 4000


def _row_mean_body(tab_ref, m_ref, out_ref):
    out_ref[...] = jnp.dot(tab_ref[...], m_ref[...],
                           preferred_element_type=jnp.float32)


def _row_means(table):
    tab2 = table.reshape(V // 2, 2 * E)
    half = jnp.concatenate(
        [jnp.ones((E, 1), jnp.float32), jnp.zeros((E, 1), jnp.float32)], axis=1)
    m = jnp.concatenate([half, half[:, ::-1]], axis=0) * (1.0 / E)
    grid = (V // 2) // _MEAN_ROWS
    out = pl.pallas_call(
        _row_mean_body,
        grid=(grid,),
        in_specs=[
            pl.BlockSpec((_MEAN_ROWS, 2 * E), lambda i: (i, 0)),
            pl.BlockSpec((2 * E, 2), lambda i: (0, 0)),
        ],
        out_specs=pl.BlockSpec((_MEAN_ROWS, 2), lambda i: (i, 0)),
        out_shape=jax.ShapeDtypeStruct((V // 2, 2), jnp.float32),
    )(tab2, m)
    return out.reshape(V)


_NC = 2
_NS = 16
_NW = _NC * _NS
_N_IDX = B * L
_PER_W = _N_IDX // _NW
_CHUNK = 128
_NCH = _PER_W // _CHUNK
_GRP = 8


def _gather_body(means_hbm, idx_hbm, out_hbm, idx_v, rows_v, sem):
    wid = lax.axis_index("s") * _NC + lax.axis_index("c")
    pltpu.sync_copy(idx_hbm.at[wid], idx_v)

    for b in range(_GRP):
        pltpu.async_copy(means_hbm.at[idx_v.at[b]], rows_v.at[b], sem)

    def body(g, carry):
        nxt = (g + 1) * _GRP
        cur = g * _GRP
        for b in range(_GRP):
            pltpu.async_copy(means_hbm.at[idx_v.at[nxt + b]], rows_v.at[nxt + b], sem)
        for b in range(_GRP):
            pltpu.make_async_copy(
                means_hbm.at[pl.ds(0, _CHUNK)], rows_v.at[cur + b], sem
            ).wait()
        return carry

    lax.fori_loop(0, _NCH // _GRP - 1, body, 0)

    last = (_NCH // _GRP - 1) * _GRP
    for b in range(_GRP):
        pltpu.make_async_copy(
            means_hbm.at[pl.ds(0, _CHUNK)], rows_v.at[last + b], sem
        ).wait()

    pltpu.sync_copy(rows_v, out_hbm.at[wid])


def _gather_means(means, idx_flat):
    idx3 = idx_flat.reshape(_NW, _NCH, _CHUNK)
    k = pl.kernel(
        _gather_body,
        out_type=jax.ShapeDtypeStruct((_NW, _NCH, _CHUNK), jnp.float32),
        mesh=plsc.VectorSubcoreMesh(core_axis_name="c", subcore_axis_name="s"),
        scratch_types=[
            pltpu.VMEM((_NCH, _CHUNK), jnp.int32),
            pltpu.VMEM((_NCH, _CHUNK), jnp.float32),
            pltpu.SemaphoreType.DMA,
        ],
    )
    return k(means, idx3).reshape(B, L)


_RB = 1024


def _img_body(img_ref, i2h_wt_ref, i2h_b_ref, out_ref):
    out_ref[...] = jnp.dot(img_ref[...], i2h_wt_ref[...],
                           preferred_element_type=jnp.float32) + i2h_b_ref[...]


def _img_hidden(image_emb, i2h_W, i2h_b):
    grid = B // _RB
    return pl.pallas_call(
        _img_body,
        grid=(grid,),
        in_specs=[
            pl.BlockSpec((_RB, IMG), lambda i: (i, 0)),
            pl.BlockSpec((IMG, H), lambda i: (0, 0)),
            pl.BlockSpec((1, H), lambda i: (0, 0)),
        ],
        out_specs=pl.BlockSpec((_RB, H), lambda i: (i, 0)),
        out_shape=jax.ShapeDtypeStruct((B, H), jnp.float32),
    )(image_emb, i2h_W.T, i2h_b.reshape(1, H))


def _final_body(qf_ref, hi_ref, q2h_wt_ref, q2h_b_ref, sc_wt_ref, sc_b_ref,
                out_ref):
    h_q = jnp.dot(qf_ref[...], q2h_wt_ref[...],
                  preferred_element_type=jnp.float32) + q2h_b_ref[...]
    comb = h_q * hi_ref[...]
    out_ref[...] = jnp.dot(comb, sc_wt_ref[...],
                           preferred_element_type=jnp.float32) + sc_b_ref[...]


def _final(qf, h_img, q2h_W, q2h_b, sc_W, sc_b):
    grid = B // _RB
    return pl.pallas_call(
        _final_body,
        grid=(grid,),
        in_specs=[
            pl.BlockSpec((_RB, L), lambda i: (i, 0)),
            pl.BlockSpec((_RB, H), lambda i: (i, 0)),
            pl.BlockSpec((L, H), lambda i: (0, 0)),
            pl.BlockSpec((1, H), lambda i: (0, 0)),
            pl.BlockSpec((H, C), lambda i: (0, 0)),
            pl.BlockSpec((1, C), lambda i: (0, 0)),
        ],
        out_specs=pl.BlockSpec((_RB, C), lambda i: (i, 0)),
        out_shape=jax.ShapeDtypeStruct((B, C), jnp.float32),
    )(qf, h_img, q2h_W.T, q2h_b.reshape(1, H), sc_W.T, sc_b.reshape(1, C))


def kernel(questions_idxs, image_emb, embs_weight, q2h_W, q2h_b, i2h_W, i2h_b,
           sc_W, sc_b):
    means = _row_means(embs_weight)
    idx_flat = questions_idxs.astype(jnp.int32).reshape(-1)
    qf = _gather_means(means, idx_flat)
    h_img = _img_hidden(image_emb, i2h_W, i2h_b)
    return _final(qf, h_img, q2h_W, q2h_b, sc_W, sc_b)


# speedup vs baseline: 4.7051x; 4.7051x over previous
import functools

import jax
import jax.numpy as jnp
from jax import lax
from jax.experimental import pallas as pl
from jax.experimental.pallas import tpu as pltpu
from jax.experimental.pallas import tpu_sc as plsc

B = 16384
L = 64
V = 1000000
E = 64
H = 128
IMG = 2048
C = 1000


_MEAN_COLS = 8192


def _row_mean_body(tab_ref, out_ref):
    out_ref[...] = jnp.mean(tab_ref[...], axis=0)


def _row_means_t(table_t):
    grid = pl.cdiv(V, _MEAN_COLS)
    return pl.pallas_call(
        _row_mean_body,
        grid=(grid,),
        in_specs=[pl.BlockSpec((E, _MEAN_COLS), lambda i: (0, i))],
        out_specs=pl.BlockSpec((_MEAN_COLS,), lambda i: (i,)),
        out_shape=jax.ShapeDtypeStruct((V,), jnp.float32),
    )(table_t)


_NC = 2
_NS = 16
_NW = _NC * _NS
_N_IDX = B * L
_PER_W = _N_IDX // _NW
_CHUNK = 128
_NCH = _PER_W // _CHUNK
_GRP = 8


def _gather_body(means_hbm, idx_hbm, out_hbm, idx_v, rows_v, sem):
    wid = lax.axis_index("s") * _NC + lax.axis_index("c")
    pltpu.sync_copy(idx_hbm.at[wid], idx_v)

    for b in range(_GRP):
        pltpu.async_copy(means_hbm.at[idx_v.at[b]], rows_v.at[b], sem)

    def body(g, carry):
        nxt = (g + 1) * _GRP
        cur = g * _GRP
        for b in range(_GRP):
            pltpu.async_copy(means_hbm.at[idx_v.at[nxt + b]], rows_v.at[nxt + b], sem)
        for b in range(_GRP):
            pltpu.make_async_copy(
                means_hbm.at[pl.ds(0, _CHUNK)], rows_v.at[cur + b], sem
            ).wait()
        return carry

    lax.fori_loop(0, _NCH // _GRP - 1, body, 0)

    last = (_NCH // _GRP - 1) * _GRP
    for b in range(_GRP):
        pltpu.make_async_copy(
            means_hbm.at[pl.ds(0, _CHUNK)], rows_v.at[last + b], sem
        ).wait()

    pltpu.sync_copy(rows_v, out_hbm.at[wid])


def _gather_means(means, idx3):
    k = pl.kernel(
        _gather_body,
        out_type=jax.ShapeDtypeStruct((_NW, _NCH, _CHUNK), jnp.float32),
        mesh=plsc.VectorSubcoreMesh(core_axis_name="c", subcore_axis_name="s"),
        scratch_types=[
            pltpu.VMEM((_NCH, _CHUNK), jnp.int32),
            pltpu.VMEM((_NCH, _CHUNK), jnp.float32),
            pltpu.SemaphoreType.DMA,
        ],
    )
    return k(means, idx3)


_RB = 1024


def _img_body(i2h_w_ref, img_ref, i2h_b_ref, out_ref):
    out_ref[...] = lax.dot_general(
        i2h_w_ref[...], img_ref[...], (((1,), (1,)), ((), ())),
        preferred_element_type=jnp.float32) + i2h_b_ref[...]


def _img_hidden_t(image_emb, i2h_W, i2h_b):
    grid = B // _RB
    return pl.pallas_call(
        _img_body,
        grid=(grid,),
        in_specs=[
            pl.BlockSpec((H, IMG), lambda i: (0, 0)),
            pl.BlockSpec((_RB, IMG), lambda i: (i, 0)),
            pl.BlockSpec((H, 1), lambda i: (0, 0)),
        ],
        out_specs=pl.BlockSpec((H, _RB), lambda i: (0, i)),
        out_shape=jax.ShapeDtypeStruct((H, B), jnp.float32),
    )(i2h_W, image_emb, i2h_b.reshape(H, 1))


def _final_body(qft_ref, hit_ref, q2h_wt_ref, q2h_b_ref, sc_w_ref, sc_b_ref,
                out_ref):
    h_qt = lax.dot_general(
        q2h_wt_ref[...], qft_ref[...], (((0,), (0,)), ((), ())),
        preferred_element_type=jnp.float32) + q2h_b_ref[...]
    comb_t = h_qt * hit_ref[...]
    out_ref[...] = lax.dot_general(
        sc_w_ref[...], comb_t, (((1,), (0,)), ((), ())),
        preferred_element_type=jnp.float32) + sc_b_ref[...]


def _final_t(qf_t, h_img_t, q2h_W, q2h_b, sc_W, sc_b):
    grid = B // _RB
    return pl.pallas_call(
        _final_body,
        grid=(grid,),
        in_specs=[
            pl.BlockSpec((L, _RB), lambda i: (0, i)),
            pl.BlockSpec((H, _RB), lambda i: (0, i)),
            pl.BlockSpec((L, H), lambda i: (0, 0)),
            pl.BlockSpec((H, 1), lambda i: (0, 0)),
            pl.BlockSpec((C, H), lambda i: (0, 0)),
            pl.BlockSpec((C, 1), lambda i: (0, 0)),
        ],
        out_specs=pl.BlockSpec((C, _RB), lambda i: (0, i)),
        out_shape=jax.ShapeDtypeStruct((C, B), jnp.float32),
    )(qf_t, h_img_t, q2h_W.T, q2h_b.reshape(H, 1), sc_W, sc_b.reshape(C, 1))


def kernel(questions_idxs, image_emb, embs_weight, q2h_W, q2h_b, i2h_W, i2h_b,
           sc_W, sc_b):
    means = _row_means_t(embs_weight.T)
    idx3 = questions_idxs.astype(jnp.int32).T.reshape(_NW, _NCH, _CHUNK)
    qf_t = _gather_means(means, idx3).reshape(L, B)
    h_img_t = _img_hidden_t(image_emb, i2h_W, i2h_b)
    logits_t = _final_t(qf_t, h_img_t, q2h_W, q2h_b, sc_W, sc_b)
    return logits_t.T


# speedup vs baseline: 5.3855x; 1.1446x over previous
import functools

import jax
import jax.numpy as jnp
from jax import lax
from jax.experimental import pallas as pl
from jax.experimental.pallas import tpu as pltpu
from jax.experimental.pallas import tpu_sc as plsc

B = 16384
L = 64
V = 1000000
E = 64
H = 128
IMG = 2048
C = 1000


_MEAN_COLS = 16384


def _row_mean_body(tab_ref, out_ref):
    out_ref[...] = jnp.mean(tab_ref[...], axis=0)


def _row_means_t(table_t):
    grid = pl.cdiv(V, _MEAN_COLS)
    return pl.pallas_call(
        _row_mean_body,
        grid=(grid,),
        in_specs=[pl.BlockSpec((E, _MEAN_COLS), lambda i: (0, i))],
        out_specs=pl.BlockSpec((_MEAN_COLS,), lambda i: (i,)),
        out_shape=jax.ShapeDtypeStruct((V,), jnp.float32),
    )(table_t)


_NC = 2
_NS = 16
_NW = _NC * _NS
_N_IDX = B * L
_PER_W = _N_IDX // _NW
_CHUNK = 128
_NCH = _PER_W // _CHUNK
_GRP = 16


def _gather_body(means_hbm, idx_hbm, out_hbm, idx_v, rows_v, sem):
    wid = lax.axis_index("s") * _NC + lax.axis_index("c")
    pltpu.sync_copy(idx_hbm.at[wid], idx_v)

    for b in range(_GRP):
        pltpu.async_copy(means_hbm.at[idx_v.at[b]], rows_v.at[b], sem)

    def body(g, carry):
        nxt = (g + 1) * _GRP
        cur = g * _GRP
        for b in range(_GRP):
            pltpu.async_copy(means_hbm.at[idx_v.at[nxt + b]], rows_v.at[nxt + b], sem)
        for b in range(_GRP):
            pltpu.make_async_copy(
                means_hbm.at[pl.ds(0, _CHUNK)], rows_v.at[cur + b], sem
            ).wait()
        return carry

    lax.fori_loop(0, _NCH // _GRP - 1, body, 0)

    last = (_NCH // _GRP - 1) * _GRP
    for b in range(_GRP):
        pltpu.make_async_copy(
            means_hbm.at[pl.ds(0, _CHUNK)], rows_v.at[last + b], sem
        ).wait()

    pltpu.sync_copy(rows_v, out_hbm.at[wid])


def _gather_means(means, idx3):
    k = pl.kernel(
        _gather_body,
        out_type=jax.ShapeDtypeStruct((_NW, _NCH, _CHUNK), jnp.float32),
        mesh=plsc.VectorSubcoreMesh(core_axis_name="c", subcore_axis_name="s"),
        scratch_types=[
            pltpu.VMEM((_NCH, _CHUNK), jnp.int32),
            pltpu.VMEM((_NCH, _CHUNK), jnp.float32),
            pltpu.SemaphoreType.DMA,
        ],
    )
    return k(means, idx3)


_RB = 1024


def _img_body(i2h_w_ref, img_ref, i2h_b_ref, out_ref):
    out_ref[...] = lax.dot_general(
        i2h_w_ref[...], img_ref[...], (((1,), (1,)), ((), ())),
        preferred_element_type=jnp.float32) + i2h_b_ref[...]


def _img_hidden_t(image_emb, i2h_W, i2h_b):
    grid = B // _RB
    return pl.pallas_call(
        _img_body,
        grid=(grid,),
        in_specs=[
            pl.BlockSpec((H, IMG), lambda i: (0, 0)),
            pl.BlockSpec((_RB, IMG), lambda i: (i, 0)),
            pl.BlockSpec((H, 1), lambda i: (0, 0)),
        ],
        out_specs=pl.BlockSpec((H, _RB), lambda i: (0, i)),
        out_shape=jax.ShapeDtypeStruct((H, B), jnp.float32),
    )(i2h_W, image_emb, i2h_b.reshape(H, 1))


def _final_body(qft_ref, hit_ref, q2h_wt_ref, q2h_b_ref, sc_w_ref, sc_b_ref,
                out_ref):
    h_qt = lax.dot_general(
        q2h_wt_ref[...], qft_ref[...], (((0,), (0,)), ((), ())),
        preferred_element_type=jnp.float32) + q2h_b_ref[...]
    comb_t = h_qt * hit_ref[...]
    out_ref[...] = lax.dot_general(
        sc_w_ref[...], comb_t, (((1,), (0,)), ((), ())),
        preferred_element_type=jnp.float32) + sc_b_ref[...]


def _final_t(qf_t, h_img_t, q2h_W, q2h_b, sc_W, sc_b):
    grid = B // _RB
    return pl.pallas_call(
        _final_body,
        grid=(grid,),
        in_specs=[
            pl.BlockSpec((L, _RB), lambda i: (0, i)),
            pl.BlockSpec((H, _RB), lambda i: (0, i)),
            pl.BlockSpec((L, H), lambda i: (0, 0)),
            pl.BlockSpec((H, 1), lambda i: (0, 0)),
            pl.BlockSpec((C, H), lambda i: (0, 0)),
            pl.BlockSpec((C, 1), lambda i: (0, 0)),
        ],
        out_specs=pl.BlockSpec((C, _RB), lambda i: (0, i)),
        out_shape=jax.ShapeDtypeStruct((C, B), jnp.float32),
    )(qf_t, h_img_t, q2h_W.T, q2h_b.reshape(H, 1), sc_W, sc_b.reshape(C, 1))


def kernel(questions_idxs, image_emb, embs_weight, q2h_W, q2h_b, i2h_W, i2h_b,
           sc_W, sc_b):
    means = _row_means_t(embs_weight.T)
    idx3 = questions_idxs.astype(jnp.int32).T.reshape(_NW, _NCH, _CHUNK)
    qf_t = _gather_means(means, idx3).reshape(L, B)
    h_img_t = _img_hidden_t(image_emb, i2h_W, i2h_b)
    logits_t = _final_t(qf_t, h_img_t, q2h_W, q2h_b, sc_W, sc_b)
    return logits_t.T


# speedup vs baseline: 5.5196x; 1.0249x over previous
import functools

import jax
import jax.numpy as jnp
from jax import lax
from jax.experimental import pallas as pl
from jax.experimental.pallas import tpu as pltpu
from jax.experimental.pallas import tpu_sc as plsc

B = 16384
L = 64
V = 1000000
E = 64
H = 128
IMG = 2048
C = 1000


_MEAN_COLS = 16384


def _row_mean_body(tab_ref, out_ref):
    out_ref[...] = jnp.mean(tab_ref[...], axis=0)


def _row_means_t(table_t):
    grid = pl.cdiv(V, _MEAN_COLS)
    return pl.pallas_call(
        _row_mean_body,
        grid=(grid,),
        in_specs=[pl.BlockSpec((E, _MEAN_COLS), lambda i: (0, i))],
        out_specs=pl.BlockSpec((_MEAN_COLS,), lambda i: (i,)),
        out_shape=jax.ShapeDtypeStruct((V,), jnp.float32),
    )(table_t)


_NC = 2
_NS = 16
_NW = _NC * _NS
_N_IDX = B * L
_PER_W = _N_IDX // _NW


def _gather_body(means_hbm, idx_hbm, out_hbm, idx_v, rows_v, sem):
    wid = lax.axis_index("s") * _NC + lax.axis_index("c")
    pltpu.sync_copy(idx_hbm.at[wid], idx_v)
    pltpu.async_copy(means_hbm.at[idx_v], rows_v, sem).wait()
    pltpu.sync_copy(rows_v, out_hbm.at[wid])


def _gather_means(means, idx2):
    k = pl.kernel(
        _gather_body,
        out_type=jax.ShapeDtypeStruct((_NW, _PER_W), jnp.float32),
        mesh=plsc.VectorSubcoreMesh(core_axis_name="c", subcore_axis_name="s"),
        scratch_types=[
            pltpu.VMEM((_PER_W,), jnp.int32),
            pltpu.VMEM((_PER_W,), jnp.float32),
            pltpu.SemaphoreType.DMA,
        ],
    )
    return k(means, idx2)


_RB = 1024


def _img_body(i2h_w_ref, img_ref, i2h_b_ref, out_ref):
    out_ref[...] = lax.dot_general(
        i2h_w_ref[...], img_ref[...], (((1,), (1,)), ((), ())),
        preferred_element_type=jnp.float32) + i2h_b_ref[...]


def _img_hidden_t(image_emb, i2h_W, i2h_b):
    grid = B // _RB
    return pl.pallas_call(
        _img_body,
        grid=(grid,),
        in_specs=[
            pl.BlockSpec((H, IMG), lambda i: (0, 0)),
            pl.BlockSpec((_RB, IMG), lambda i: (i, 0)),
            pl.BlockSpec((H, 1), lambda i: (0, 0)),
        ],
        out_specs=pl.BlockSpec((H, _RB), lambda i: (0, i)),
        out_shape=jax.ShapeDtypeStruct((H, B), jnp.float32),
    )(i2h_W, image_emb, i2h_b.reshape(H, 1))


def _final_body(qft_ref, hit_ref, q2h_wt_ref, q2h_b_ref, sc_w_ref, sc_b_ref,
                out_ref):
    h_qt = lax.dot_general(
        q2h_wt_ref[...], qft_ref[...], (((0,), (0,)), ((), ())),
        preferred_element_type=jnp.float32) + q2h_b_ref[...]
    comb_t = h_qt * hit_ref[...]
    out_ref[...] = lax.dot_general(
        sc_w_ref[...], comb_t, (((1,), (0,)), ((), ())),
        preferred_element_type=jnp.float32) + sc_b_ref[...]


def _final_t(qf_t, h_img_t, q2h_W, q2h_b, sc_W, sc_b):
    grid = B // _RB
    return pl.pallas_call(
        _final_body,
        grid=(grid,),
        in_specs=[
            pl.BlockSpec((L, _RB), lambda i: (0, i)),
            pl.BlockSpec((H, _RB), lambda i: (0, i)),
            pl.BlockSpec((L, H), lambda i: (0, 0)),
            pl.BlockSpec((H, 1), lambda i: (0, 0)),
            pl.BlockSpec((C, H), lambda i: (0, 0)),
            pl.BlockSpec((C, 1), lambda i: (0, 0)),
        ],
        out_specs=pl.BlockSpec((C, _RB), lambda i: (0, i)),
        out_shape=jax.ShapeDtypeStruct((C, B), jnp.float32),
    )(qf_t, h_img_t, q2h_W.T, q2h_b.reshape(H, 1), sc_W, sc_b.reshape(C, 1))


def kernel(questions_idxs, image_emb, embs_weight, q2h_W, q2h_b, i2h_W, i2h_b,
           sc_W, sc_b):
    means = _row_means_t(embs_weight.T)
    idx2 = questions_idxs.astype(jnp.int32).T.reshape(_NW, _PER_W)
    qf_t = _gather_means(means, idx2).reshape(L, B)
    h_img_t = _img_hidden_t(image_emb, i2h_W, i2h_b)
    logits_t = _final_t(qf_t, h_img_t, q2h_W, q2h_b, sc_W, sc_b)
    return logits_t.T


# speedup vs baseline: 5.6400x; 1.0218x over previous
import functools

import jax
import jax.numpy as jnp
from jax import lax
from jax.experimental import pallas as pl
from jax.experimental.pallas import tpu as pltpu
from jax.experimental.pallas import tpu_sc as plsc

B = 16384
L = 64
V = 1000000
E = 64
H = 128
IMG = 2048
C = 1000


_MEAN_COLS = 16384


def _row_mean_body(tab_ref, out_ref):
    out_ref[...] = jnp.mean(tab_ref[...], axis=0)


def _row_means_t(table_t):
    grid = pl.cdiv(V, _MEAN_COLS)
    return pl.pallas_call(
        _row_mean_body,
        grid=(grid,),
        in_specs=[pl.BlockSpec((E, _MEAN_COLS), lambda i: (0, i))],
        out_specs=pl.BlockSpec((_MEAN_COLS,), lambda i: (i,)),
        out_shape=jax.ShapeDtypeStruct((V,), jnp.float32),
    )(table_t)


_NC = 2
_NS = 16
_NW = _NC * _NS
_N_IDX = B * L
_PER_W = _N_IDX // _NW


def _gather_body(means_hbm, idx_hbm, out_hbm, idx_v, rows_v, sem):
    wid = lax.axis_index("s") * _NC + lax.axis_index("c")
    pltpu.sync_copy(idx_hbm.at[wid], idx_v)
    pltpu.async_copy(means_hbm.at[idx_v], rows_v, sem).wait()
    pltpu.sync_copy(rows_v, out_hbm.at[wid])


def _gather_means(means, idx2):
    k = pl.kernel(
        _gather_body,
        out_type=jax.ShapeDtypeStruct((_NW, _PER_W), jnp.float32),
        mesh=plsc.VectorSubcoreMesh(core_axis_name="c", subcore_axis_name="s"),
        scratch_types=[
            pltpu.VMEM((_PER_W,), jnp.int32),
            pltpu.VMEM((_PER_W,), jnp.float32),
            pltpu.SemaphoreType.DMA,
        ],
    )
    return k(means, idx2)


_RB = 2048


def _img_body(i2h_w_ref, img_ref, i2h_b_ref, out_ref):
    out_ref[...] = lax.dot_general(
        i2h_w_ref[...], img_ref[...], (((1,), (1,)), ((), ())),
        preferred_element_type=jnp.float32) + i2h_b_ref[...]


def _img_hidden_t(image_emb, i2h_W, i2h_b):
    grid = B // _RB
    return pl.pallas_call(
        _img_body,
        grid=(grid,),
        in_specs=[
            pl.BlockSpec((H, IMG), lambda i: (0, 0)),
            pl.BlockSpec((_RB, IMG), lambda i: (i, 0)),
            pl.BlockSpec((H, 1), lambda i: (0, 0)),
        ],
        out_specs=pl.BlockSpec((H, _RB), lambda i: (0, i)),
        out_shape=jax.ShapeDtypeStruct((H, B), jnp.float32),
    )(i2h_W, image_emb, i2h_b.reshape(H, 1))


def _final_body(qft_ref, hit_ref, q2h_wt_ref, q2h_b_ref, sc_w_ref, sc_b_ref,
                out_ref):
    h_qt = lax.dot_general(
        q2h_wt_ref[...], qft_ref[...], (((0,), (0,)), ((), ())),
        preferred_element_type=jnp.float32) + q2h_b_ref[...]
    comb_t = h_qt * hit_ref[...]
    out_ref[...] = lax.dot_general(
        sc_w_ref[...], comb_t, (((1,), (0,)), ((), ())),
        preferred_element_type=jnp.float32) + sc_b_ref[...]


def _final_t(qf_t, h_img_t, q2h_W, q2h_b, sc_W, sc_b):
    grid = B // _RB
    return pl.pallas_call(
        _final_body,
        grid=(grid,),
        in_specs=[
            pl.BlockSpec((L, _RB), lambda i: (0, i)),
            pl.BlockSpec((H, _RB), lambda i: (0, i)),
            pl.BlockSpec((L, H), lambda i: (0, 0)),
            pl.BlockSpec((H, 1), lambda i: (0, 0)),
            pl.BlockSpec((C, H), lambda i: (0, 0)),
            pl.BlockSpec((C, 1), lambda i: (0, 0)),
        ],
        out_specs=pl.BlockSpec((C, _RB), lambda i: (0, i)),
        out_shape=jax.ShapeDtypeStruct((C, B), jnp.float32),
    )(qf_t, h_img_t, q2h_W.T, q2h_b.reshape(H, 1), sc_W, sc_b.reshape(C, 1))


def kernel(questions_idxs, image_emb, embs_weight, q2h_W, q2h_b, i2h_W, i2h_b,
           sc_W, sc_b):
    means = _row_means_t(embs_weight.T)
    idx2 = questions_idxs.astype(jnp.int32).T.reshape(_NW, _PER_W)
    qf_t = _gather_means(means, idx2).reshape(L, B)
    h_img_t = _img_hidden_t(image_emb, i2h_W, i2h_b)
    logits_t = _final_t(qf_t, h_img_t, q2h_W, q2h_b, sc_W, sc_b)
    return logits_t.T


# speedup vs baseline: 6.1304x; 1.0870x over previous
import functools

import jax
import jax.numpy as jnp
from jax import lax
from jax.experimental import pallas as pl
from jax.experimental.pallas import tpu as pltpu
from jax.experimental.pallas import tpu_sc as plsc

B = 16384
L = 64
V = 1000000
E = 64
H = 128
IMG = 2048
C = 1000


_MEAN_COLS = 16384


def _row_mean_body(tab_ref, out_ref):
    out_ref[...] = jnp.mean(tab_ref[...], axis=0)


_V_PAD = 1000192


def _row_means_t(table_t):
    grid = pl.cdiv(V, _MEAN_COLS)
    return pl.pallas_call(
        _row_mean_body,
        grid=(grid,),
        in_specs=[pl.BlockSpec((E, _MEAN_COLS), lambda i: (0, i))],
        out_specs=pl.BlockSpec((_MEAN_COLS,), lambda i: (i,)),
        out_shape=jax.ShapeDtypeStruct((_V_PAD,), jnp.float32),
    )(table_t)


_NC = 2
_NS = 16
_NW = _NC * _NS
_N_IDX = B * L
_PER_W = _N_IDX // _NW


_SEG = _V_PAD // _NS
_HOP = _SEG // 2


def _gather_body(means_hbm, idx_hbm, out_hbm, idx_v, rows_v, sem, means_sh):
    sid = lax.axis_index("s")
    wid = sid * _NC + lax.axis_index("c")
    for h in range(2):
        off = sid * _SEG + h * _HOP
        pltpu.sync_copy(means_hbm.at[pl.ds(off, _HOP)],
                        rows_v.at[pl.ds(0, _HOP)])
        pltpu.sync_copy(rows_v.at[pl.ds(0, _HOP)],
                        means_sh.at[pl.ds(off, _HOP)])
    pltpu.sync_copy(idx_hbm.at[wid], idx_v)
    plsc.subcore_barrier()
    pltpu.async_copy(means_sh.at[idx_v], rows_v, sem).wait()
    pltpu.sync_copy(rows_v, out_hbm.at[wid])


def _gather_means(means, idx2):
    k = pl.kernel(
        _gather_body,
        out_type=jax.ShapeDtypeStruct((_NW, _PER_W), jnp.float32),
        mesh=plsc.VectorSubcoreMesh(core_axis_name="c", subcore_axis_name="s"),
        scratch_types=[
            pltpu.VMEM((_PER_W,), jnp.int32),
            pltpu.VMEM((_PER_W,), jnp.float32),
            pltpu.SemaphoreType.DMA,
            pltpu.VMEM_SHARED((_V_PAD,), jnp.float32),
        ],
    )
    return k(means, idx2)


_RB = 2048


def _img_body(i2h_w_ref, img_ref, i2h_b_ref, out_ref):
    out_ref[...] = lax.dot_general(
        i2h_w_ref[...], img_ref[...], (((1,), (1,)), ((), ())),
        preferred_element_type=jnp.float32) + i2h_b_ref[...]


def _img_hidden_t(image_emb, i2h_W, i2h_b):
    grid = B // _RB
    return pl.pallas_call(
        _img_body,
        grid=(grid,),
        in_specs=[
            pl.BlockSpec((H, IMG), lambda i: (0, 0)),
            pl.BlockSpec((_RB, IMG), lambda i: (i, 0)),
            pl.BlockSpec((H, 1), lambda i: (0, 0)),
        ],
        out_specs=pl.BlockSpec((H, _RB), lambda i: (0, i)),
        out_shape=jax.ShapeDtypeStruct((H, B), jnp.float32),
    )(i2h_W, image_emb, i2h_b.reshape(H, 1))


def _final_body(qft_ref, hit_ref, q2h_wt_ref, q2h_b_ref, sc_w_ref, sc_b_ref,
                out_ref):
    h_qt = lax.dot_general(
        q2h_wt_ref[...], qft_ref[...], (((0,), (0,)), ((), ())),
        preferred_element_type=jnp.float32) + q2h_b_ref[...]
    comb_t = h_qt * hit_ref[...]
    out_ref[...] = lax.dot_general(
        sc_w_ref[...], comb_t, (((1,), (0,)), ((), ())),
        preferred_element_type=jnp.float32) + sc_b_ref[...]


def _final_t(qf_t, h_img_t, q2h_W, q2h_b, sc_W, sc_b):
    grid = B // _RB
    return pl.pallas_call(
        _final_body,
        grid=(grid,),
        in_specs=[
            pl.BlockSpec((L, _RB), lambda i: (0, i)),
            pl.BlockSpec((H, _RB), lambda i: (0, i)),
            pl.BlockSpec((L, H), lambda i: (0, 0)),
            pl.BlockSpec((H, 1), lambda i: (0, 0)),
            pl.BlockSpec((C, H), lambda i: (0, 0)),
            pl.BlockSpec((C, 1), lambda i: (0, 0)),
        ],
        out_specs=pl.BlockSpec((C, _RB), lambda i: (0, i)),
        out_shape=jax.ShapeDtypeStruct((C, B), jnp.float32),
    )(qf_t, h_img_t, q2h_W.T, q2h_b.reshape(H, 1), sc_W, sc_b.reshape(C, 1))


def kernel(questions_idxs, image_emb, embs_weight, q2h_W, q2h_b, i2h_W, i2h_b,
           sc_W, sc_b):
    means = _row_means_t(embs_weight.T)
    idx2 = questions_idxs.astype(jnp.int32).T.reshape(_NW, _PER_W)
    qf_t = _gather_means(means, idx2).reshape(L, B)
    h_img_t = _img_hidden_t(image_emb, i2h_W, i2h_b)
    logits_t = _final_t(qf_t, h_img_t, q2h_W, q2h_b, sc_W, sc_b)
    return logits_t.T


# speedup vs baseline: 6.3246x; 1.0317x over previous
import functools

import jax
import jax.numpy as jnp
from jax import lax
from jax.experimental import pallas as pl
from jax.experimental.pallas import tpu as pltpu
from jax.experimental.pallas import tpu_sc as plsc

B = 16384
L = 64
V = 1000000
E = 64
H = 128
IMG = 2048
C = 1000


_MEAN_COLS = 32768


def _row_mean_body(tab_ref, out_ref):
    out_ref[...] = jnp.mean(tab_ref[...], axis=0)


_V_PAD = 1000192


def _row_means_t(table_t):
    grid = pl.cdiv(V, _MEAN_COLS)
    return pl.pallas_call(
        _row_mean_body,
        grid=(grid,),
        in_specs=[pl.BlockSpec((E, _MEAN_COLS), lambda i: (0, i))],
        out_specs=pl.BlockSpec((_MEAN_COLS,), lambda i: (i,)),
        out_shape=jax.ShapeDtypeStruct((_V_PAD,), jnp.float32),
    )(table_t)


_NC = 2
_NS = 16
_NW = _NC * _NS
_N_IDX = B * L
_PER_W = _N_IDX // _NW


_SEG = _V_PAD // _NS
_HOP = _SEG // 2


def _gather_body(means_hbm, idx_hbm, out_hbm, idx_v, rows_v, sem, means_sh):
    sid = lax.axis_index("s")
    wid = sid * _NC + lax.axis_index("c")
    for h in range(2):
        off = sid * _SEG + h * _HOP
        pltpu.sync_copy(means_hbm.at[pl.ds(off, _HOP)],
                        rows_v.at[pl.ds(0, _HOP)])
        pltpu.sync_copy(rows_v.at[pl.ds(0, _HOP)],
                        means_sh.at[pl.ds(off, _HOP)])
    pltpu.sync_copy(idx_hbm.at[wid], idx_v)
    plsc.subcore_barrier()
    pltpu.async_copy(means_sh.at[idx_v], rows_v, sem).wait()
    pltpu.sync_copy(rows_v, out_hbm.at[wid])


def _gather_means(means, idx2):
    k = pl.kernel(
        _gather_body,
        out_type=jax.ShapeDtypeStruct((_NW, _PER_W), jnp.float32),
        mesh=plsc.VectorSubcoreMesh(core_axis_name="c", subcore_axis_name="s"),
        scratch_types=[
            pltpu.VMEM((_PER_W,), jnp.int32),
            pltpu.VMEM((_PER_W,), jnp.float32),
            pltpu.SemaphoreType.DMA,
            pltpu.VMEM_SHARED((_V_PAD,), jnp.float32),
        ],
    )
    return k(means, idx2)


_RB = 2048


def _img_body(i2h_w_ref, img_ref, i2h_b_ref, out_ref):
    out_ref[...] = lax.dot_general(
        i2h_w_ref[...], img_ref[...], (((1,), (1,)), ((), ())),
        preferred_element_type=jnp.float32) + i2h_b_ref[...]


def _img_hidden_t(image_emb, i2h_W, i2h_b):
    grid = B // _RB
    return pl.pallas_call(
        _img_body,
        grid=(grid,),
        in_specs=[
            pl.BlockSpec((H, IMG), lambda i: (0, 0)),
            pl.BlockSpec((_RB, IMG), lambda i: (i, 0)),
            pl.BlockSpec((H, 1), lambda i: (0, 0)),
        ],
        out_specs=pl.BlockSpec((H, _RB), lambda i: (0, i)),
        out_shape=jax.ShapeDtypeStruct((H, B), jnp.float32),
    )(i2h_W, image_emb, i2h_b.reshape(H, 1))


def _final_body(qft_ref, hit_ref, q2h_wt_ref, q2h_b_ref, sc_w_ref, sc_b_ref,
                out_ref):
    h_qt = lax.dot_general(
        q2h_wt_ref[...], qft_ref[...], (((0,), (0,)), ((), ())),
        preferred_element_type=jnp.float32) + q2h_b_ref[...]
    comb_t = h_qt * hit_ref[...]
    out_ref[...] = lax.dot_general(
        sc_w_ref[...], comb_t, (((1,), (0,)), ((), ())),
        preferred_element_type=jnp.float32) + sc_b_ref[...]


def _final_t(qf_t, h_img_t, q2h_W, q2h_b, sc_W, sc_b):
    grid = B // _RB
    return pl.pallas_call(
        _final_body,
        grid=(grid,),
        in_specs=[
            pl.BlockSpec((L, _RB), lambda i: (0, i)),
            pl.BlockSpec((H, _RB), lambda i: (0, i)),
            pl.BlockSpec((L, H), lambda i: (0, 0)),
            pl.BlockSpec((H, 1), lambda i: (0, 0)),
            pl.BlockSpec((C, H), lambda i: (0, 0)),
            pl.BlockSpec((C, 1), lambda i: (0, 0)),
        ],
        out_specs=pl.BlockSpec((C, _RB), lambda i: (0, i)),
        out_shape=jax.ShapeDtypeStruct((C, B), jnp.float32),
    )(qf_t, h_img_t, q2h_W.T, q2h_b.reshape(H, 1), sc_W, sc_b.reshape(C, 1))


def kernel(questions_idxs, image_emb, embs_weight, q2h_W, q2h_b, i2h_W, i2h_b,
           sc_W, sc_b):
    means = _row_means_t(embs_weight.T)
    idx2 = questions_idxs.astype(jnp.int32).T.reshape(_NW, _PER_W)
    qf_t = _gather_means(means, idx2).reshape(L, B)
    h_img_t = _img_hidden_t(image_emb, i2h_W, i2h_b)
    logits_t = _final_t(qf_t, h_img_t, q2h_W, q2h_b, sc_W, sc_b)
    return logits_t.T


# speedup vs baseline: 6.3543x; 1.0047x over previous
import jax
import jax.numpy as jnp
from jax import lax
from jax.experimental import pallas as pl
from jax.experimental.pallas import tpu as pltpu
from jax.experimental.pallas import tpu_sc as plsc

B = 16384
L = 64
V = 1000000
E = 64
H = 128
IMG = 2048
C = 1000


_MEAN_COLS = 32768


def _row_mean_body(tab_ref, out_ref):
    out_ref[...] = jnp.mean(tab_ref[...], axis=0)


_V_PAD = 1000192


def _row_means_t(table_t):
    grid = pl.cdiv(V, _MEAN_COLS)
    return pl.pallas_call(
        _row_mean_body,
        grid=(grid,),
        in_specs=[pl.BlockSpec((E, _MEAN_COLS), lambda i: (0, i))],
        out_specs=pl.BlockSpec((_MEAN_COLS,), lambda i: (i,)),
        out_shape=jax.ShapeDtypeStruct((_V_PAD,), jnp.float32),
    )(table_t)


_NC = 2
_NS = 16
_NW = _NC * _NS
_N_IDX = B * L
_PER_W = _N_IDX // _NW


_SEG = _V_PAD // _NS
_HOP = _SEG // 2


def _gather_body(means_hbm, idx_hbm, out_hbm, idx_v, rows_v, sem, means_sh):
    sid = lax.axis_index("s")
    wid = sid * _NC + lax.axis_index("c")
    for h in range(2):
        off = sid * _SEG + h * _HOP
        pltpu.sync_copy(means_hbm.at[pl.ds(off, _HOP)],
                        rows_v.at[pl.ds(0, _HOP)])
        pltpu.sync_copy(rows_v.at[pl.ds(0, _HOP)],
                        means_sh.at[pl.ds(off, _HOP)])
    pltpu.sync_copy(idx_hbm.at[wid], idx_v)
    plsc.subcore_barrier()
    pltpu.async_copy(means_sh.at[idx_v], rows_v, sem).wait()
    pltpu.sync_copy(rows_v, out_hbm.at[wid])


def _gather_means(means, idx2):
    k = pl.kernel(
        _gather_body,
        out_type=jax.ShapeDtypeStruct((_NW, _PER_W), jnp.float32),
        mesh=plsc.VectorSubcoreMesh(core_axis_name="c", subcore_axis_name="s"),
        scratch_types=[
            pltpu.VMEM((_PER_W,), jnp.int32),
            pltpu.VMEM((_PER_W,), jnp.float32),
            pltpu.SemaphoreType.DMA,
            pltpu.VMEM_SHARED((_V_PAD,), jnp.float32),
        ],
    )
    return k(means, idx2)


_RB = 2048


def _img_body(i2h_w_ref, img_ref, i2h_b_ref, out_ref):
    out_ref[...] = lax.dot_general(
        i2h_w_ref[...], img_ref[...], (((1,), (1,)), ((), ())),
        preferred_element_type=jnp.float32) + i2h_b_ref[...]


def _img_hidden_t(image_emb, i2h_W, i2h_b):
    grid = B // _RB
    return pl.pallas_call(
        _img_body,
        grid=(grid,),
        in_specs=[
            pl.BlockSpec((H, IMG), lambda i: (0, 0)),
            pl.BlockSpec((_RB, IMG), lambda i: (i, 0)),
            pl.BlockSpec((H, 1), lambda i: (0, 0)),
        ],
        out_specs=pl.BlockSpec((H, _RB), lambda i: (0, i)),
        out_shape=jax.ShapeDtypeStruct((H, B), jnp.float32),
    )(i2h_W, image_emb, i2h_b.reshape(H, 1))


def _final_body(qft_ref, hit_ref, q2h_wt_ref, q2h_b_ref, sc_w_ref, sc_b_ref,
                out_ref):
    h_qt = lax.dot_general(
        q2h_wt_ref[...], qft_ref[...], (((0,), (0,)), ((), ())),
        preferred_element_type=jnp.float32) + q2h_b_ref[...]
    comb_t = h_qt * hit_ref[...]
    out_ref[...] = lax.dot_general(
        sc_w_ref[...], comb_t, (((1,), (0,)), ((), ())),
        preferred_element_type=jnp.float32) + sc_b_ref[...]


def _final_t(qf_t, h_img_t, q2h_W, q2h_b, sc_W, sc_b):
    grid = B // _RB
    return pl.pallas_call(
        _final_body,
        grid=(grid,),
        in_specs=[
            pl.BlockSpec((L, _RB), lambda i: (0, i)),
            pl.BlockSpec((H, _RB), lambda i: (0, i)),
            pl.BlockSpec((L, H), lambda i: (0, 0)),
            pl.BlockSpec((H, 1), lambda i: (0, 0)),
            pl.BlockSpec((C, H), lambda i: (0, 0)),
            pl.BlockSpec((C, 1), lambda i: (0, 0)),
        ],
        out_specs=pl.BlockSpec((C, _RB), lambda i: (0, i)),
        out_shape=jax.ShapeDtypeStruct((C, B), jnp.float32),
    )(qf_t, h_img_t, q2h_W.T, q2h_b.reshape(H, 1), sc_W, sc_b.reshape(C, 1))


def kernel(questions_idxs, image_emb, embs_weight, q2h_W, q2h_b, i2h_W, i2h_b,
           sc_W, sc_b):
    means = _row_means_t(embs_weight.T)
    idx2 = questions_idxs.astype(jnp.int32).T.reshape(_NW, _PER_W)
    qf_t = _gather_means(means, idx2).reshape(L, B)
    h_img_t = _img_hidden_t(image_emb, i2h_W, i2h_b)
    logits_t = _final_t(qf_t, h_img_t, q2h_W, q2h_b, sc_W, sc_b)
    return logits_t.T
